# baseline (speedup 1.0000x reference)
import jax
import jax.numpy as jnp
from jax import lax
from jax.experimental import pallas as pl
from jax.experimental.pallas import tpu as pltpu

CTX = 200
PATH = 20
EMBED = 64
PATH_PAD = 32
NBUF = 4
CHUNK = CTX // NBUF


def _body(ctx_idx_ref, path_idx_ref, bits_ref, ctx_table_ref, node_table_ref,
          o_ref, c0, c1, c2, c3, nrows, sem, nsem):
    bufs = [c0, c1, c2, c3]
    for i in range(CTX):
        pltpu.make_async_copy(
            ctx_table_ref.at[pl.ds(ctx_idx_ref[i], 1)],
            bufs[i % NBUF].at[pl.ds(i // NBUF, 1)], sem.at[i % NBUF]).start()
    for i in range(PATH_PAD):
        pltpu.make_async_copy(
            node_table_ref.at[pl.ds(path_idx_ref[i], 1)],
            nrows.at[pl.ds(i, 1)], nsem).start()

    for q in range(NBUF):
        def drain(i, _, q=q):
            pltpu.make_async_copy(
                node_table_ref.at[pl.ds(0, 1)], bufs[q].at[pl.ds(0, 1)],
                sem.at[q]).wait()
            return 0
        lax.fori_loop(0, CHUNK, drain, 0)

    def drain_n(i, _):
        pltpu.make_async_copy(
            node_table_ref.at[pl.ds(0, 1)], nrows.at[pl.ds(0, 1)],
            nsem).wait()
        return 0
    lax.fori_loop(0, PATH_PAD, drain_n, 0)

    acc = (jnp.sum(c0[...], axis=0, keepdims=True)
           + jnp.sum(c1[...], axis=0, keepdims=True)
           + jnp.sum(c2[...], axis=0, keepdims=True)
           + jnp.sum(c3[...], axis=0, keepdims=True))
    h = acc * (1.0 / CTX)
    n = nrows[...]
    b = bits_ref[...]
    t = jnp.sum(n * h, axis=1, keepdims=True)
    s = jax.nn.sigmoid(t)
    eps = 1e-9
    per = -b * jnp.log(s + eps) - (1.0 - b) * jnp.log(1.0 - s + eps)
    row = lax.broadcasted_iota(jnp.int32, (PATH_PAD, 1), 0)
    per = jnp.where(row < PATH, per, 0.0)
    o_ref[0, 0] = jnp.sum(per)


_call = pl.pallas_call(
    _body,
    in_specs=[
        pl.BlockSpec(memory_space=pltpu.SMEM),
        pl.BlockSpec(memory_space=pltpu.SMEM),
        pl.BlockSpec(memory_space=pltpu.VMEM),
        pl.BlockSpec(memory_space=pl.ANY),
        pl.BlockSpec(memory_space=pl.ANY),
    ],
    out_specs=pl.BlockSpec(memory_space=pltpu.SMEM),
    out_shape=jax.ShapeDtypeStruct((1, 1), jnp.float32),
    scratch_shapes=[
        pltpu.VMEM((CHUNK, EMBED), jnp.float32),
        pltpu.VMEM((CHUNK, EMBED), jnp.float32),
        pltpu.VMEM((CHUNK, EMBED), jnp.float32),
        pltpu.VMEM((CHUNK, EMBED), jnp.float32),
        pltpu.VMEM((PATH_PAD, EMBED), jnp.float32),
        pltpu.SemaphoreType.DMA((NBUF,)),
        pltpu.SemaphoreType.DMA,
    ],
)


def kernel(context_idx, path_indices, code_bits, context_table, node_table):
    ctx = jnp.asarray(context_idx, jnp.int32)
    pidx = jnp.asarray(path_indices, jnp.int32)
    path_pad = jnp.zeros((PATH_PAD,), jnp.int32).at[:PATH].set(pidx)
    bits_col = (jnp.zeros((PATH_PAD, 1), jnp.float32)
                .at[:PATH, 0].set(code_bits.astype(jnp.float32)))
    out = _call(ctx, path_pad, bits_col, context_table, node_table)
    return out[0, 0]
